# Initial kernel scaffold; baseline (speedup 1.0000x reference)
import jax
import jax.numpy as jnp
from jax.experimental import pallas as pl


def kernel(x, W_q, bn_gamma, bn_beta, keys, down_embed, up_embed):
    raise NotImplementedError("write your pallas kernel here")


# speedup vs baseline: 6.8996x; 6.8996x over previous
import functools

import jax
import jax.numpy as jnp
from jax import lax
from jax.experimental import pallas as pl
from jax.experimental.pallas import tpu as pltpu
from jax.experimental.pallas import tpu_sc as plsc

_B, _S, _D = 2, 2048, 2048
_H = 8
_KD = 128
_NE = 16384
_NK = 128
_TOPK = 2
_N = _B * _S
_F = 2 * _H * _KD
_TM = 256
_NT = _N // _TM
_EPS = 1e-5

_NC, _NS = 2, 16
_NW = _NC * _NS
_TPW = _N // _NW
_PK = _H * _TOPK
_LANES = 16
_CHUNKS = _D // _LANES


def _proj_body(x_ref, w_ref, q_ref, s1_ref, s2_ref):
    i = pl.program_id(0)
    q = lax.dot_general(x_ref[...], w_ref[...], (((1,), (1,)), ((), ())),
                        preferred_element_type=jnp.float32)
    q_ref[...] = q

    @pl.when(i == 0)
    def _init():
        s1_ref[...] = jnp.zeros_like(s1_ref)
        s2_ref[...] = jnp.zeros_like(s2_ref)

    s1_ref[...] += jnp.sum(q, axis=0, keepdims=True)
    s2_ref[...] += jnp.sum(q * q, axis=0, keepdims=True)


def _route_body(q_ref, s1_ref, s2_ref, g_ref, b_ref, k_ref, idx_ref, w_ref):
    n = jnp.float32(_N)
    mean = s1_ref[...] / n
    var = s2_ref[...] / n - mean * mean
    qn = (q_ref[...] - mean) / jnp.sqrt(var + _EPS) * g_ref[...] + b_ref[...]
    idx_cols = []
    w_cols = []
    for h in range(_H):
        per_p = []
        for p in range(2):
            f0 = p * _H * _KD + h * _KD
            qs = qn[:, f0:f0 + _KD]
            kp = k_ref[p * _H + h]
            sim = lax.dot_general(qs, kp, (((1,), (1,)), ((), ())),
                                  preferred_element_type=jnp.float32)
            iota = lax.broadcasted_iota(jnp.int32, sim.shape, 1)
            m1 = jnp.max(sim, axis=1, keepdims=True)
            i1 = jnp.min(jnp.where(sim == m1, iota, _NK), axis=1, keepdims=True)
            simm = jnp.where(iota == i1, -jnp.inf, sim)
            m2 = jnp.max(simm, axis=1, keepdims=True)
            i2 = jnp.min(jnp.where(simm == m2, iota, _NK), axis=1, keepdims=True)
            per_p.append((m1, i1, m2, i2))
        (m1x, i1x, m2x, i2x), (m1y, i1y, m2y, i2y) = per_p
        s_a = m1x + m1y
        idx_a = i1x * _NK + i1y
        c01 = m1x + m2y
        c10 = m2x + m1y
        take01 = c01 >= c10
        s_b = jnp.where(take01, c01, c10)
        idx_b = jnp.where(take01, i1x * _NK + i2y, i2x * _NK + i1y)
        e = jnp.exp(s_b - s_a)
        denom = 1.0 + e
        idx_cols += [idx_a, idx_b]
        w_cols += [1.0 / denom, e / denom]
    idx_ref[...] = jnp.concatenate(idx_cols, axis=1)
    w_ref[...] = jnp.concatenate(w_cols, axis=1)


def _make_sc_combine():
    mesh = plsc.VectorSubcoreMesh(core_axis_name="c", subcore_axis_name="s",
                                  num_cores=_NC, num_subcores=_NS)

    @functools.partial(
        pl.kernel,
        out_type=jax.ShapeDtypeStruct((_N, _D), jnp.float32),
        mesh=mesh,
        compiler_params=pltpu.CompilerParams(needs_layout_passes=False),
        scratch_types=[
            pltpu.VMEM((_TPW * _PK,), jnp.int32),
            pltpu.VMEM((_TPW * _PK,), jnp.float32),
            pltpu.VMEM((_D,), jnp.float32),
            pltpu.VMEM((_PK, _D), jnp.float32),
            pltpu.VMEM((_PK, _D), jnp.float32),
            pltpu.VMEM((_D,), jnp.float32),
            pltpu.SemaphoreType.DMA,
            pltpu.SemaphoreType.DMA,
        ],
    )
    def _sc(x_hbm, idx_hbm, w_hbm, down_hbm, up_hbm, out_hbm,
            idx_v, w_v, xbuf, gd, gu, obuf, semd, semu):
        wid = lax.axis_index("s") * _NC + lax.axis_index("c")
        base = wid * _TPW
        pltpu.sync_copy(idx_hbm.at[pl.ds(base * _PK, _TPW * _PK)], idx_v)
        pltpu.sync_copy(w_hbm.at[pl.ds(base * _PK, _TPW * _PK)], w_v)

        def token_body(t, carry):
            tok = base + t
            pltpu.sync_copy(x_hbm.at[tok], xbuf)
            isl = idx_v.at[pl.ds(t * _PK, _PK)]
            cp_d = pltpu.async_copy(down_hbm.at[isl], gd, semd)
            cp_u = pltpu.async_copy(up_hbm.at[isl], gu, semu)
            cp_d.wait()

            def dot_chunk(j, accs):
                sl = pl.ds(pl.multiple_of(j * _LANES, _LANES), _LANES)
                xc = xbuf[sl]
                return tuple(accs[k] + xc * gd[k, sl] for k in range(_PK))

            accs = lax.fori_loop(
                0, _CHUNKS, dot_chunk,
                tuple(jnp.zeros((_LANES,), jnp.float32) for _ in range(_PK)))
            lane = lax.iota(jnp.int32, _LANES)
            hv = jnp.zeros((_LANES,), jnp.float32)
            for k in range(_PK):
                hv = jnp.where(lane == k, jnp.sum(accs[k]), hv)
            wv = w_v[pl.ds(t * _PK, _PK)]
            u = 0.7978845608028654 * (hv + 0.044715 * hv * hv * hv)
            e2u = jnp.exp(2.0 * u)
            tanh_u = 1.0 - 2.0 / (e2u + 1.0)
            coeff = wv * (0.5 * hv * (1.0 + tanh_u))
            cs = [jnp.sum(jnp.where(lane == k, coeff, 0.0)) for k in range(_PK)]
            cp_u.wait()

            def up_chunk(j, carry2):
                sl = pl.ds(pl.multiple_of(j * _LANES, _LANES), _LANES)
                o = cs[0] * gu[0, sl]
                for k in range(1, _PK):
                    o = o + cs[k] * gu[k, sl]
                obuf[sl] = o
                return carry2

            lax.fori_loop(0, _CHUNKS, up_chunk, 0)
            pltpu.sync_copy(obuf, out_hbm.at[tok])
            return carry

        lax.fori_loop(0, _TPW, token_body, 0)

    return _sc


def kernel(x, W_q, bn_gamma, bn_beta, keys, down_embed, up_embed):
    xf = x.reshape(_N, _D)
    q, s1, s2 = pl.pallas_call(
        _proj_body,
        grid=(_NT,),
        in_specs=[pl.BlockSpec((_TM, _D), lambda i: (i, 0)),
                  pl.BlockSpec((_F, _D), lambda i: (0, 0))],
        out_specs=[pl.BlockSpec((_TM, _F), lambda i: (i, 0)),
                   pl.BlockSpec((1, _F), lambda i: (0, 0)),
                   pl.BlockSpec((1, _F), lambda i: (0, 0))],
        out_shape=[jax.ShapeDtypeStruct((_N, _F), jnp.float32),
                   jax.ShapeDtypeStruct((1, _F), jnp.float32),
                   jax.ShapeDtypeStruct((1, _F), jnp.float32)],
    )(xf, W_q)

    keys_r = jnp.transpose(keys, (2, 0, 1, 3)).reshape(2 * _H, _NK, _KD)
    g2 = bn_gamma.reshape(1, _F)
    b2 = bn_beta.reshape(1, _F)
    idxs, ws = pl.pallas_call(
        _route_body,
        grid=(_NT,),
        in_specs=[pl.BlockSpec((_TM, _F), lambda i: (i, 0)),
                  pl.BlockSpec((1, _F), lambda i: (0, 0)),
                  pl.BlockSpec((1, _F), lambda i: (0, 0)),
                  pl.BlockSpec((1, _F), lambda i: (0, 0)),
                  pl.BlockSpec((1, _F), lambda i: (0, 0)),
                  pl.BlockSpec((2 * _H, _NK, _KD), lambda i: (0, 0, 0))],
        out_specs=[pl.BlockSpec((_TM, _PK), lambda i: (i, 0)),
                   pl.BlockSpec((_TM, _PK), lambda i: (i, 0))],
        out_shape=[jax.ShapeDtypeStruct((_N, _PK), jnp.int32),
                   jax.ShapeDtypeStruct((_N, _PK), jnp.float32)],
    )(q, s1, s2, g2, b2, keys_r)

    out = _make_sc_combine()(xf, idxs.reshape(-1), ws.reshape(-1),
                             down_embed, up_embed)
    return out.reshape(_B, _S, _D)


# speedup vs baseline: 10.1901x; 1.4769x over previous
import functools

import jax
import jax.numpy as jnp
from jax import lax
from jax.experimental import pallas as pl
from jax.experimental.pallas import tpu as pltpu
from jax.experimental.pallas import tpu_sc as plsc

_B, _S, _D = 2, 2048, 2048
_H = 8
_KD = 128
_NE = 16384
_NK = 128
_TOPK = 2
_N = _B * _S
_F = 2 * _H * _KD
_TM = 256
_NT = _N // _TM
_EPS = 1e-5

_NC, _NS = 2, 16
_NW = _NC * _NS
_TPW = _N // _NW
_PK = _H * _TOPK
_LANES = 16
_CHUNKS = _D // _LANES


def _proj_body(x_ref, w_ref, q_ref, s1_ref, s2_ref):
    i = pl.program_id(0)
    q = lax.dot_general(x_ref[...], w_ref[...], (((1,), (1,)), ((), ())),
                        preferred_element_type=jnp.float32)
    q_ref[...] = q

    @pl.when(i == 0)
    def _init():
        s1_ref[...] = jnp.zeros_like(s1_ref)
        s2_ref[...] = jnp.zeros_like(s2_ref)

    s1_ref[...] += jnp.sum(q, axis=0, keepdims=True)
    s2_ref[...] += jnp.sum(q * q, axis=0, keepdims=True)


def _route_body(q_ref, s1_ref, s2_ref, g_ref, b_ref, k_ref, idx_ref, w_ref):
    n = jnp.float32(_N)
    mean = s1_ref[...] / n
    var = s2_ref[...] / n - mean * mean
    qn = (q_ref[...] - mean) / jnp.sqrt(var + _EPS) * g_ref[...] + b_ref[...]
    idx_cols = []
    w_cols = []
    for h in range(_H):
        per_p = []
        for p in range(2):
            f0 = p * _H * _KD + h * _KD
            qs = qn[:, f0:f0 + _KD]
            kp = k_ref[p * _H + h]
            sim = lax.dot_general(qs, kp, (((1,), (1,)), ((), ())),
                                  preferred_element_type=jnp.float32)
            iota = lax.broadcasted_iota(jnp.int32, sim.shape, 1)
            m1 = jnp.max(sim, axis=1, keepdims=True)
            i1 = jnp.min(jnp.where(sim == m1, iota, _NK), axis=1, keepdims=True)
            simm = jnp.where(iota == i1, -jnp.inf, sim)
            m2 = jnp.max(simm, axis=1, keepdims=True)
            i2 = jnp.min(jnp.where(simm == m2, iota, _NK), axis=1, keepdims=True)
            per_p.append((m1, i1, m2, i2))
        (m1x, i1x, m2x, i2x), (m1y, i1y, m2y, i2y) = per_p
        s_a = m1x + m1y
        idx_a = i1x * _NK + i1y
        c01 = m1x + m2y
        c10 = m2x + m1y
        take01 = c01 >= c10
        s_b = jnp.where(take01, c01, c10)
        idx_b = jnp.where(take01, i1x * _NK + i2y, i2x * _NK + i1y)
        e = jnp.exp(s_b - s_a)
        denom = 1.0 + e
        idx_cols += [idx_a, idx_b]
        w_cols += [1.0 / denom, e / denom]
    idx_ref[...] = jnp.concatenate(idx_cols, axis=1)
    w_ref[...] = jnp.concatenate(w_cols, axis=1)


_HALF = _PK // 2


def _make_sc_combine():
    mesh = plsc.VectorSubcoreMesh(core_axis_name="c", subcore_axis_name="s",
                                  num_cores=_NC, num_subcores=_NS)

    @functools.partial(
        pl.kernel,
        out_type=jax.ShapeDtypeStruct((_N, _D), jnp.float32),
        mesh=mesh,
        compiler_params=pltpu.CompilerParams(needs_layout_passes=False),
        scratch_types=[
            pltpu.VMEM((_TPW * _PK,), jnp.int32),
            pltpu.VMEM((_TPW * _PK,), jnp.float32),
            pltpu.VMEM((2 * _D,), jnp.float32),
            pltpu.VMEM((_HALF, _D), jnp.float32),
            pltpu.VMEM((_HALF, _D), jnp.float32),
            pltpu.VMEM((_HALF, _D), jnp.float32),
            pltpu.VMEM((_HALF, _D), jnp.float32),
            pltpu.VMEM((_D,), jnp.float32),
            pltpu.SemaphoreType.DMA,
            pltpu.SemaphoreType.DMA,
            pltpu.SemaphoreType.DMA,
            pltpu.SemaphoreType.DMA,
            pltpu.SemaphoreType.DMA,
            pltpu.SemaphoreType.DMA,
        ],
    )
    def _sc(x_hbm, idx_hbm, w_hbm, down_hbm, up_hbm, out_hbm,
            idx_v, w_v, xbuf, gd0, gd1, gu0, gu1, obuf,
            semx, semd0, semu0, semd1, semu1, semo):
        wid = lax.axis_index("s") * _NC + lax.axis_index("c")
        base = wid * _TPW
        pltpu.sync_copy(idx_hbm.at[pl.ds(base * _PK, _TPW * _PK)], idx_v)
        pltpu.sync_copy(w_hbm.at[pl.ds(base * _PK, _TPW * _PK)], w_v)
        lane = lax.iota(jnp.int32, _LANES)

        def issue_half(t, half, gd, gu, semd, semu):
            isl = idx_v.at[pl.ds(t * _PK + half * _HALF, _HALF)]
            pltpu.async_copy(down_hbm.at[isl], gd, semd)
            pltpu.async_copy(up_hbm.at[isl], gu, semu)

        def wait_g(tab_hbm, g, sem):
            pltpu.make_async_copy(tab_hbm.at[pl.ds(0, _HALF)], g, sem).wait()

        def dot_half(xoff, gd):
            def dot_chunk(j, accs):
                o = pl.multiple_of(j * _LANES, _LANES)
                xc = xbuf[pl.ds(xoff + o, _LANES)]
                return tuple(accs[k] + xc * gd[k, pl.ds(o, _LANES)]
                             for k in range(_HALF))
            return lax.fori_loop(
                0, _CHUNKS, dot_chunk,
                tuple(jnp.zeros((_LANES,), jnp.float32) for _ in range(_HALF)))

        def coeffs(accs, wv, half):
            hv = jnp.zeros((_LANES,), jnp.float32)
            for k in range(_HALF):
                hv = jnp.where(lane == half * _HALF + k, jnp.sum(accs[k]), hv)
            u = 0.7978845608028654 * (hv + 0.044715 * hv * hv * hv)
            e2u = jnp.exp(2.0 * u)
            tanh_u = 1.0 - 2.0 / (e2u + 1.0)
            coeff = wv * (0.5 * hv * (1.0 + tanh_u))
            return [jnp.sum(jnp.where(lane == half * _HALF + k, coeff, 0.0))
                    for k in range(_HALF)]

        pltpu.async_copy(x_hbm.at[base], xbuf.at[pl.ds(0, _D)], semx)
        issue_half(0, 0, gd0, gu0, semd0, semu0)

        def token_body(t, carry):
            tok = base + t
            xoff = (t % 2) * _D
            issue_half(t, 1, gd1, gu1, semd1, semu1)
            pltpu.make_async_copy(x_hbm.at[tok], xbuf.at[pl.ds(xoff, _D)],
                                  semx).wait()

            @pl.when(t < _TPW - 1)
            def _():
                pltpu.async_copy(x_hbm.at[tok + 1],
                                 xbuf.at[pl.ds(_D - xoff, _D)], semx)

            wv = w_v[pl.ds(t * _PK, _PK)]
            wait_g(down_hbm, gd0, semd0)
            cs0 = coeffs(dot_half(xoff, gd0), wv, 0)

            @pl.when(t > 0)
            def _():
                pltpu.make_async_copy(obuf, out_hbm.at[tok], semo).wait()

            wait_g(up_hbm, gu0, semu0)

            def up0(j, c2):
                s = pl.ds(pl.multiple_of(j * _LANES, _LANES), _LANES)
                v = cs0[0] * gu0[0, s]
                for k in range(1, _HALF):
                    v = v + cs0[k] * gu0[k, s]
                obuf[s] = v
                return c2

            lax.fori_loop(0, _CHUNKS, up0, 0)

            @pl.when(t < _TPW - 1)
            def _():
                issue_half(t + 1, 0, gd0, gu0, semd0, semu0)

            wait_g(down_hbm, gd1, semd1)
            cs1 = coeffs(dot_half(xoff, gd1), wv, 1)
            wait_g(up_hbm, gu1, semu1)

            def up1(j, c2):
                s = pl.ds(pl.multiple_of(j * _LANES, _LANES), _LANES)
                v = obuf[s]
                for k in range(_HALF):
                    v = v + cs1[k] * gu1[k, s]
                obuf[s] = v
                return c2

            lax.fori_loop(0, _CHUNKS, up1, 0)
            pltpu.async_copy(obuf, out_hbm.at[tok], semo)
            return carry

        lax.fori_loop(0, _TPW, token_body, 0)
        pltpu.make_async_copy(obuf, out_hbm.at[base], semo).wait()

    return _sc


def kernel(x, W_q, bn_gamma, bn_beta, keys, down_embed, up_embed):
    xf = x.reshape(_N, _D)
    q, s1, s2 = pl.pallas_call(
        _proj_body,
        grid=(_NT,),
        in_specs=[pl.BlockSpec((_TM, _D), lambda i: (i, 0)),
                  pl.BlockSpec((_F, _D), lambda i: (0, 0))],
        out_specs=[pl.BlockSpec((_TM, _F), lambda i: (i, 0)),
                   pl.BlockSpec((1, _F), lambda i: (0, 0)),
                   pl.BlockSpec((1, _F), lambda i: (0, 0))],
        out_shape=[jax.ShapeDtypeStruct((_N, _F), jnp.float32),
                   jax.ShapeDtypeStruct((1, _F), jnp.float32),
                   jax.ShapeDtypeStruct((1, _F), jnp.float32)],
    )(xf, W_q)

    keys_r = jnp.transpose(keys, (2, 0, 1, 3)).reshape(2 * _H, _NK, _KD)
    g2 = bn_gamma.reshape(1, _F)
    b2 = bn_beta.reshape(1, _F)
    idxs, ws = pl.pallas_call(
        _route_body,
        grid=(_NT,),
        in_specs=[pl.BlockSpec((_TM, _F), lambda i: (i, 0)),
                  pl.BlockSpec((1, _F), lambda i: (0, 0)),
                  pl.BlockSpec((1, _F), lambda i: (0, 0)),
                  pl.BlockSpec((1, _F), lambda i: (0, 0)),
                  pl.BlockSpec((1, _F), lambda i: (0, 0)),
                  pl.BlockSpec((2 * _H, _NK, _KD), lambda i: (0, 0, 0))],
        out_specs=[pl.BlockSpec((_TM, _PK), lambda i: (i, 0)),
                   pl.BlockSpec((_TM, _PK), lambda i: (i, 0))],
        out_shape=[jax.ShapeDtypeStruct((_N, _PK), jnp.int32),
                   jax.ShapeDtypeStruct((_N, _PK), jnp.float32)],
    )(q, s1, s2, g2, b2, keys_r)

    out = _make_sc_combine()(xf, idxs.reshape(-1), ws.reshape(-1),
                             down_embed, up_embed)
    return out.reshape(_B, _S, _D)


# speedup vs baseline: 11.5415x; 1.1326x over previous
import functools

import jax
import jax.numpy as jnp
from jax import lax
from jax.experimental import pallas as pl
from jax.experimental.pallas import tpu as pltpu
from jax.experimental.pallas import tpu_sc as plsc

_B, _S, _D = 2, 2048, 2048
_H = 8
_KD = 128
_NE = 16384
_NK = 128
_TOPK = 2
_N = _B * _S
_F = 2 * _H * _KD
_TM = 256
_NT = _N // _TM
_EPS = 1e-5

_NC, _NS = 2, 16
_NW = _NC * _NS
_TPW = _N // _NW
_PK = _H * _TOPK
_LANES = 16
_CHUNKS = _D // _LANES


def _proj_body(x_ref, w_ref, q_ref, s1_ref, s2_ref):
    i = pl.program_id(0)
    q = lax.dot_general(x_ref[...], w_ref[...], (((1,), (1,)), ((), ())),
                        preferred_element_type=jnp.float32)
    q_ref[...] = q

    @pl.when(i == 0)
    def _init():
        s1_ref[...] = jnp.zeros_like(s1_ref)
        s2_ref[...] = jnp.zeros_like(s2_ref)

    s1_ref[...] += jnp.sum(q, axis=0, keepdims=True)
    s2_ref[...] += jnp.sum(q * q, axis=0, keepdims=True)


def _route_body(q_ref, s1_ref, s2_ref, g_ref, b_ref, k_ref, idx_ref, w_ref):
    n = jnp.float32(_N)
    mean = s1_ref[...] / n
    var = s2_ref[...] / n - mean * mean
    qn = (q_ref[...] - mean) / jnp.sqrt(var + _EPS) * g_ref[...] + b_ref[...]
    idx_cols = []
    w_cols = []
    for h in range(_H):
        per_p = []
        for p in range(2):
            f0 = p * _H * _KD + h * _KD
            qs = qn[:, f0:f0 + _KD]
            kp = k_ref[p * _H + h]
            sim = lax.dot_general(qs, kp, (((1,), (1,)), ((), ())),
                                  preferred_element_type=jnp.float32)
            iota = lax.broadcasted_iota(jnp.int32, sim.shape, 1)
            m1 = jnp.max(sim, axis=1, keepdims=True)
            i1 = jnp.min(jnp.where(sim == m1, iota, _NK), axis=1, keepdims=True)
            simm = jnp.where(iota == i1, -jnp.inf, sim)
            m2 = jnp.max(simm, axis=1, keepdims=True)
            i2 = jnp.min(jnp.where(simm == m2, iota, _NK), axis=1, keepdims=True)
            per_p.append((m1, i1, m2, i2))
        (m1x, i1x, m2x, i2x), (m1y, i1y, m2y, i2y) = per_p
        s_a = m1x + m1y
        idx_a = i1x * _NK + i1y
        c01 = m1x + m2y
        c10 = m2x + m1y
        take01 = c01 >= c10
        s_b = jnp.where(take01, c01, c10)
        idx_b = jnp.where(take01, i1x * _NK + i2y, i2x * _NK + i1y)
        e = jnp.exp(s_b - s_a)
        denom = 1.0 + e
        idx_cols += [idx_a, idx_b]
        w_cols += [1.0 / denom, e / denom]
    idx_ref[...] = jnp.concatenate(idx_cols, axis=1)
    w_ref[...] = jnp.concatenate(w_cols, axis=1)


_HALF = _PK // 2
_HD = _D // 2
_CH32 = _HD // _LANES
_HIMASK = -65536


def _pack_bf16(v):
    bits = lax.bitcast_convert_type(v, jnp.int32)
    r = bits + 0x7FFF + ((bits >> 16) & 1)
    lo = (r[:, :_HD] >> 16) & 0xFFFF
    hi = r[:, _HD:] & _HIMASK
    return hi | lo


def _cast_body(d_ref, u_ref, do_ref, uo_ref):
    do_ref[...] = _pack_bf16(d_ref[...])
    uo_ref[...] = _pack_bf16(u_ref[...])


def _make_sc_combine():
    mesh = plsc.VectorSubcoreMesh(core_axis_name="c", subcore_axis_name="s",
                                  num_cores=_NC, num_subcores=_NS)

    @functools.partial(
        pl.kernel,
        out_type=jax.ShapeDtypeStruct((_N, _D), jnp.float32),
        mesh=mesh,
        compiler_params=pltpu.CompilerParams(needs_layout_passes=False),
        scratch_types=[
            pltpu.VMEM((_TPW * _PK,), jnp.int32),
            pltpu.VMEM((_TPW * _PK,), jnp.float32),
            pltpu.VMEM((2 * _D,), jnp.float32),
            pltpu.VMEM((_HALF, _HD), jnp.int32),
            pltpu.VMEM((_HALF, _HD), jnp.int32),
            pltpu.VMEM((_HALF, _HD), jnp.int32),
            pltpu.VMEM((_HALF, _HD), jnp.int32),
            pltpu.VMEM((_D,), jnp.float32),
            pltpu.SemaphoreType.DMA,
            pltpu.SemaphoreType.DMA,
            pltpu.SemaphoreType.DMA,
            pltpu.SemaphoreType.DMA,
            pltpu.SemaphoreType.DMA,
            pltpu.SemaphoreType.DMA,
        ],
    )
    def _sc(x_hbm, idx_hbm, w_hbm, down_hbm, up_hbm, out_hbm,
            idx_v, w_v, xbuf, gd0, gd1, gu0, gu1, obuf,
            semx, semd0, semu0, semd1, semu1, semo):
        wid = lax.axis_index("s") * _NC + lax.axis_index("c")
        base = wid * _TPW
        pltpu.sync_copy(idx_hbm.at[pl.ds(base * _PK, _TPW * _PK)], idx_v)
        pltpu.sync_copy(w_hbm.at[pl.ds(base * _PK, _TPW * _PK)], w_v)
        lane = lax.iota(jnp.int32, _LANES)

        def issue_half(t, half, gd, gu, semd, semu):
            isl = idx_v.at[pl.ds(t * _PK + half * _HALF, _HALF)]
            pltpu.async_copy(down_hbm.at[isl], gd, semd)
            pltpu.async_copy(up_hbm.at[isl], gu, semu)

        def wait_g(tab_hbm, g, sem):
            pltpu.make_async_copy(tab_hbm.at[pl.ds(0, _HALF)], g, sem).wait()

        def unpair(ci):
            lo = plsc.bitcast(ci << 16, jnp.float32)
            hi = plsc.bitcast(ci & _HIMASK, jnp.float32)
            return lo, hi

        def dot_half(xoff, gd):
            def dot_chunk(j, accs):
                o = pl.multiple_of(j * _LANES, _LANES)
                xe = xbuf[pl.ds(xoff + o, _LANES)]
                xo = xbuf[pl.ds(xoff + _HD + o, _LANES)]
                new = []
                for k in range(_HALF):
                    lo, hi = unpair(gd[k, pl.ds(o, _LANES)])
                    new.append(accs[k] + xe * lo + xo * hi)
                return tuple(new)
            return lax.fori_loop(
                0, _CH32, dot_chunk,
                tuple(jnp.zeros((_LANES,), jnp.float32) for _ in range(_HALF)))

        def coeffs(accs, wv, half):
            hv = jnp.zeros((_LANES,), jnp.float32)
            for k in range(_HALF):
                hv = jnp.where(lane == half * _HALF + k, jnp.sum(accs[k]), hv)
            u = 0.7978845608028654 * (hv + 0.044715 * hv * hv * hv)
            e2u = jnp.exp(2.0 * u)
            tanh_u = 1.0 - 2.0 / (e2u + 1.0)
            coeff = wv * (0.5 * hv * (1.0 + tanh_u))
            return [jnp.sum(jnp.where(lane == half * _HALF + k, coeff, 0.0))
                    for k in range(_HALF)]

        pltpu.async_copy(x_hbm.at[base], xbuf.at[pl.ds(0, _D)], semx)
        issue_half(0, 0, gd0, gu0, semd0, semu0)

        def token_body(t, carry):
            tok = base + t
            xoff = (t % 2) * _D
            issue_half(t, 1, gd1, gu1, semd1, semu1)
            pltpu.make_async_copy(x_hbm.at[tok], xbuf.at[pl.ds(xoff, _D)],
                                  semx).wait()

            @pl.when(t < _TPW - 1)
            def _():
                pltpu.async_copy(x_hbm.at[tok + 1],
                                 xbuf.at[pl.ds(_D - xoff, _D)], semx)

            wv = w_v[pl.ds(t * _PK, _PK)]
            wait_g(down_hbm, gd0, semd0)
            cs0 = coeffs(dot_half(xoff, gd0), wv, 0)

            @pl.when(t > 0)
            def _():
                pltpu.make_async_copy(obuf, out_hbm.at[tok], semo).wait()

            wait_g(up_hbm, gu0, semu0)

            def up0(j, c2):
                o = pl.multiple_of(j * _LANES, _LANES)
                lo, hi = unpair(gu0[0, pl.ds(o, _LANES)])
                ve = cs0[0] * lo
                vo = cs0[0] * hi
                for k in range(1, _HALF):
                    lo, hi = unpair(gu0[k, pl.ds(o, _LANES)])
                    ve = ve + cs0[k] * lo
                    vo = vo + cs0[k] * hi
                obuf[pl.ds(o, _LANES)] = ve
                obuf[pl.ds(_HD + o, _LANES)] = vo
                return c2

            lax.fori_loop(0, _CH32, up0, 0)

            @pl.when(t < _TPW - 1)
            def _():
                issue_half(t + 1, 0, gd0, gu0, semd0, semu0)

            wait_g(down_hbm, gd1, semd1)
            cs1 = coeffs(dot_half(xoff, gd1), wv, 1)
            wait_g(up_hbm, gu1, semu1)

            def up1(j, c2):
                o = pl.multiple_of(j * _LANES, _LANES)
                ve = obuf[pl.ds(o, _LANES)]
                vo = obuf[pl.ds(_HD + o, _LANES)]
                for k in range(_HALF):
                    lo, hi = unpair(gu1[k, pl.ds(o, _LANES)])
                    ve = ve + cs1[k] * lo
                    vo = vo + cs1[k] * hi
                obuf[pl.ds(o, _LANES)] = ve
                obuf[pl.ds(_HD + o, _LANES)] = vo
                return c2

            lax.fori_loop(0, _CH32, up1, 0)
            pltpu.async_copy(obuf, out_hbm.at[tok], semo)
            return carry

        lax.fori_loop(0, _TPW, token_body, 0)
        pltpu.make_async_copy(obuf, out_hbm.at[base], semo).wait()

    return _sc


def kernel(x, W_q, bn_gamma, bn_beta, keys, down_embed, up_embed):
    xf = x.reshape(_N, _D)
    q, s1, s2 = pl.pallas_call(
        _proj_body,
        grid=(_NT,),
        in_specs=[pl.BlockSpec((_TM, _D), lambda i: (i, 0)),
                  pl.BlockSpec((_F, _D), lambda i: (0, 0))],
        out_specs=[pl.BlockSpec((_TM, _F), lambda i: (i, 0)),
                   pl.BlockSpec((1, _F), lambda i: (0, 0)),
                   pl.BlockSpec((1, _F), lambda i: (0, 0))],
        out_shape=[jax.ShapeDtypeStruct((_N, _F), jnp.float32),
                   jax.ShapeDtypeStruct((1, _F), jnp.float32),
                   jax.ShapeDtypeStruct((1, _F), jnp.float32)],
    )(xf, W_q)

    keys_r = jnp.transpose(keys, (2, 0, 1, 3)).reshape(2 * _H, _NK, _KD)
    g2 = bn_gamma.reshape(1, _F)
    b2 = bn_beta.reshape(1, _F)
    idxs, ws = pl.pallas_call(
        _route_body,
        grid=(_NT,),
        in_specs=[pl.BlockSpec((_TM, _F), lambda i: (i, 0)),
                  pl.BlockSpec((1, _F), lambda i: (0, 0)),
                  pl.BlockSpec((1, _F), lambda i: (0, 0)),
                  pl.BlockSpec((1, _F), lambda i: (0, 0)),
                  pl.BlockSpec((1, _F), lambda i: (0, 0)),
                  pl.BlockSpec((2 * _H, _NK, _KD), lambda i: (0, 0, 0))],
        out_specs=[pl.BlockSpec((_TM, _PK), lambda i: (i, 0)),
                   pl.BlockSpec((_TM, _PK), lambda i: (i, 0))],
        out_shape=[jax.ShapeDtypeStruct((_N, _PK), jnp.int32),
                   jax.ShapeDtypeStruct((_N, _PK), jnp.float32)],
    )(q, s1, s2, g2, b2, keys_r)

    cast_rows = 512
    down_pk, up_pk = pl.pallas_call(
        _cast_body,
        grid=(_NE // cast_rows,),
        in_specs=[pl.BlockSpec((cast_rows, _D), lambda i: (i, 0)),
                  pl.BlockSpec((cast_rows, _D), lambda i: (i, 0))],
        out_specs=[pl.BlockSpec((cast_rows, _HD), lambda i: (i, 0)),
                   pl.BlockSpec((cast_rows, _HD), lambda i: (i, 0))],
        out_shape=[jax.ShapeDtypeStruct((_NE, _HD), jnp.int32),
                   jax.ShapeDtypeStruct((_NE, _HD), jnp.int32)],
    )(down_embed, up_embed)

    out = _make_sc_combine()(xf, idxs.reshape(-1), ws.reshape(-1),
                             down_pk, up_pk)
    return out.reshape(_B, _S, _D)


# speedup vs baseline: 12.3985x; 1.0743x over previous
import functools

import jax
import jax.numpy as jnp
from jax import lax
from jax.experimental import pallas as pl
from jax.experimental.pallas import tpu as pltpu
from jax.experimental.pallas import tpu_sc as plsc

_B, _S, _D = 2, 2048, 2048
_H = 8
_KD = 128
_NE = 16384
_NK = 128
_TOPK = 2
_N = _B * _S
_F = 2 * _H * _KD
_TM = 256
_NT = _N // _TM
_EPS = 1e-5

_TMA = 128
_NTA = _N // _TMA
_CAST_ROWS = _NE // _NTA

_NC, _NS = 2, 16
_NW = _NC * _NS
_TPW = _N // _NW
_PK = _H * _TOPK
_LANES = 16
_CHUNKS = _D // _LANES


_HD = _D // 2
_HIMASK = -65536


def _pack_bf16(v):
    bits = lax.bitcast_convert_type(v, jnp.int32)
    r = bits + 0x7FFF + ((bits >> 16) & 1)
    lo = (r[:, :_HD] >> 16) & 0xFFFF
    hi = r[:, _HD:] & _HIMASK
    return hi | lo


def _proj_body(x_ref, w_ref, d_ref, u_ref, q_ref, s1_ref, s2_ref,
               do_ref, uo_ref):
    i = pl.program_id(0)
    q = lax.dot_general(x_ref[...], w_ref[...], (((1,), (1,)), ((), ())),
                        preferred_element_type=jnp.float32)
    q_ref[...] = q

    @pl.when(i == 0)
    def _init():
        s1_ref[...] = jnp.zeros_like(s1_ref)
        s2_ref[...] = jnp.zeros_like(s2_ref)

    s1_ref[...] += jnp.sum(q, axis=0, keepdims=True)
    s2_ref[...] += jnp.sum(q * q, axis=0, keepdims=True)
    do_ref[...] = _pack_bf16(d_ref[...])
    uo_ref[...] = _pack_bf16(u_ref[...])


def _route_body(q_ref, s1_ref, s2_ref, g_ref, b_ref, k_ref, idx_ref, w_ref):
    n = jnp.float32(_N)
    mean = s1_ref[...] / n
    var = s2_ref[...] / n - mean * mean
    qn = (q_ref[...] - mean) / jnp.sqrt(var + _EPS) * g_ref[...] + b_ref[...]
    iota = lax.broadcasted_iota(jnp.int32, (_TM, _NK), 1).astype(jnp.float32)
    idx_cols = []
    w_cols = []
    for h in range(_H):
        per_p = []
        for p in range(2):
            f0 = p * _H * _KD + h * _KD
            qs = qn[:, f0:f0 + _KD]
            kp = k_ref[p * _H + h]
            sim = lax.dot_general(qs, kp, (((1,), (1,)), ((), ())),
                                  preferred_element_type=jnp.float32)
            m1 = jnp.max(sim, axis=1, keepdims=True)
            i1 = jnp.min(jnp.where(sim == m1, iota, jnp.float32(_NK)),
                         axis=1, keepdims=True)
            simm = jnp.where(iota == i1, -jnp.inf, sim)
            m2 = jnp.max(simm, axis=1, keepdims=True)
            i2 = jnp.min(jnp.where(simm == m2, iota, jnp.float32(_NK)),
                         axis=1, keepdims=True)
            per_p.append((m1, i1, m2, i2))
        (m1x, i1x, m2x, i2x), (m1y, i1y, m2y, i2y) = per_p
        s_a = m1x + m1y
        idx_a = i1x * _NK + i1y
        c01 = m1x + m2y
        c10 = m2x + m1y
        take01 = c01 >= c10
        s_b = jnp.where(take01, c01, c10)
        idx_b = jnp.where(take01, i1x * _NK + i2y, i2x * _NK + i1y)
        e = jnp.exp(s_b - s_a)
        denom = 1.0 + e
        idx_cols += [idx_a, idx_b]
        w_cols += [1.0 / denom, e / denom]
    idx_ref[...] = jnp.concatenate(idx_cols, axis=1).astype(jnp.int32)
    w_ref[...] = jnp.concatenate(w_cols, axis=1)


_HALF = _PK // 2
_CH32 = _HD // _LANES


def _make_sc_combine():
    mesh = plsc.VectorSubcoreMesh(core_axis_name="c", subcore_axis_name="s",
                                  num_cores=_NC, num_subcores=_NS)

    @functools.partial(
        pl.kernel,
        out_type=jax.ShapeDtypeStruct((_N, _D), jnp.float32),
        mesh=mesh,
        compiler_params=pltpu.CompilerParams(needs_layout_passes=False),
        scratch_types=[
            pltpu.VMEM((_TPW * _PK,), jnp.int32),
            pltpu.VMEM((_TPW * _PK,), jnp.float32),
            pltpu.VMEM((2 * _D,), jnp.float32),
            pltpu.VMEM((_HALF, _HD), jnp.int32),
            pltpu.VMEM((_HALF, _HD), jnp.int32),
            pltpu.VMEM((_HALF, _HD), jnp.int32),
            pltpu.VMEM((_HALF, _HD), jnp.int32),
            pltpu.VMEM((_D,), jnp.float32),
            pltpu.SemaphoreType.DMA,
            pltpu.SemaphoreType.DMA,
            pltpu.SemaphoreType.DMA,
            pltpu.SemaphoreType.DMA,
            pltpu.SemaphoreType.DMA,
            pltpu.SemaphoreType.DMA,
        ],
    )
    def _sc(x_hbm, idx_hbm, w_hbm, down_hbm, up_hbm, out_hbm,
            idx_v, w_v, xbuf, gd0, gd1, gu0, gu1, obuf,
            semx, semd0, semu0, semd1, semu1, semo):
        wid = lax.axis_index("s") * _NC + lax.axis_index("c")
        base = wid * _TPW
        pltpu.sync_copy(idx_hbm.at[pl.ds(base * _PK, _TPW * _PK)], idx_v)
        pltpu.sync_copy(w_hbm.at[pl.ds(base * _PK, _TPW * _PK)], w_v)
        lane = lax.iota(jnp.int32, _LANES)

        def issue_half(t, half, gd, gu, semd, semu):
            isl = idx_v.at[pl.ds(t * _PK + half * _HALF, _HALF)]
            pltpu.async_copy(down_hbm.at[isl], gd, semd)
            pltpu.async_copy(up_hbm.at[isl], gu, semu)

        def wait_g(tab_hbm, g, sem):
            pltpu.make_async_copy(tab_hbm.at[pl.ds(0, _HALF)], g, sem).wait()

        def unpair(ci):
            lo = plsc.bitcast(ci << 16, jnp.float32)
            hi = plsc.bitcast(ci & _HIMASK, jnp.float32)
            return lo, hi

        def dot_half(xoff, gd):
            def dot_chunk(j, accs):
                o = pl.multiple_of(j * _LANES, _LANES)
                xe = xbuf[pl.ds(xoff + o, _LANES)]
                xo = xbuf[pl.ds(xoff + _HD + o, _LANES)]
                new = []
                for k in range(_HALF):
                    lo, hi = unpair(gd[k, pl.ds(o, _LANES)])
                    new.append(accs[k] + xe * lo + xo * hi)
                return tuple(new)
            return lax.fori_loop(
                0, _CH32, dot_chunk,
                tuple(jnp.zeros((_LANES,), jnp.float32) for _ in range(_HALF)))

        def coeffs(accs, wv, half):
            hv = jnp.zeros((_LANES,), jnp.float32)
            for k in range(_HALF):
                hv = jnp.where(lane == half * _HALF + k, jnp.sum(accs[k]), hv)
            u = 0.7978845608028654 * (hv + 0.044715 * hv * hv * hv)
            e2u = jnp.exp(2.0 * u)
            tanh_u = 1.0 - 2.0 / (e2u + 1.0)
            coeff = wv * (0.5 * hv * (1.0 + tanh_u))
            return [jnp.sum(jnp.where(lane == half * _HALF + k, coeff, 0.0))
                    for k in range(_HALF)]

        pltpu.async_copy(x_hbm.at[base], xbuf.at[pl.ds(0, _D)], semx)
        issue_half(0, 0, gd0, gu0, semd0, semu0)

        def token_body(t, carry):
            tok = base + t
            xoff = (t % 2) * _D
            issue_half(t, 1, gd1, gu1, semd1, semu1)
            pltpu.make_async_copy(x_hbm.at[tok], xbuf.at[pl.ds(xoff, _D)],
                                  semx).wait()

            @pl.when(t < _TPW - 1)
            def _():
                pltpu.async_copy(x_hbm.at[tok + 1],
                                 xbuf.at[pl.ds(_D - xoff, _D)], semx)

            wv = w_v[pl.ds(t * _PK, _PK)]
            wait_g(down_hbm, gd0, semd0)
            cs0 = coeffs(dot_half(xoff, gd0), wv, 0)

            @pl.when(t > 0)
            def _():
                pltpu.make_async_copy(obuf, out_hbm.at[tok], semo).wait()

            wait_g(up_hbm, gu0, semu0)

            def up0(j, c2):
                o = pl.multiple_of(j * _LANES, _LANES)
                lo, hi = unpair(gu0[0, pl.ds(o, _LANES)])
                ve = cs0[0] * lo
                vo = cs0[0] * hi
                for k in range(1, _HALF):
                    lo, hi = unpair(gu0[k, pl.ds(o, _LANES)])
                    ve = ve + cs0[k] * lo
                    vo = vo + cs0[k] * hi
                obuf[pl.ds(o, _LANES)] = ve
                obuf[pl.ds(_HD + o, _LANES)] = vo
                return c2

            lax.fori_loop(0, _CH32, up0, 0)

            @pl.when(t < _TPW - 1)
            def _():
                issue_half(t + 1, 0, gd0, gu0, semd0, semu0)

            wait_g(down_hbm, gd1, semd1)
            cs1 = coeffs(dot_half(xoff, gd1), wv, 1)
            wait_g(up_hbm, gu1, semu1)

            def up1(j, c2):
                o = pl.multiple_of(j * _LANES, _LANES)
                ve = obuf[pl.ds(o, _LANES)]
                vo = obuf[pl.ds(_HD + o, _LANES)]
                for k in range(_HALF):
                    lo, hi = unpair(gu1[k, pl.ds(o, _LANES)])
                    ve = ve + cs1[k] * lo
                    vo = vo + cs1[k] * hi
                obuf[pl.ds(o, _LANES)] = ve
                obuf[pl.ds(_HD + o, _LANES)] = vo
                return c2

            lax.fori_loop(0, _CH32, up1, 0)
            pltpu.async_copy(obuf, out_hbm.at[tok], semo)
            return carry

        lax.fori_loop(0, _TPW, token_body, 0)
        pltpu.make_async_copy(obuf, out_hbm.at[base], semo).wait()

    return _sc


def kernel(x, W_q, bn_gamma, bn_beta, keys, down_embed, up_embed):
    xf = x.reshape(_N, _D)
    q, s1, s2, down_pk, up_pk = pl.pallas_call(
        _proj_body,
        grid=(_NTA,),
        in_specs=[pl.BlockSpec((_TMA, _D), lambda i: (i, 0)),
                  pl.BlockSpec((_F, _D), lambda i: (0, 0)),
                  pl.BlockSpec((_CAST_ROWS, _D), lambda i: (i, 0)),
                  pl.BlockSpec((_CAST_ROWS, _D), lambda i: (i, 0))],
        out_specs=[pl.BlockSpec((_TMA, _F), lambda i: (i, 0)),
                   pl.BlockSpec((1, _F), lambda i: (0, 0)),
                   pl.BlockSpec((1, _F), lambda i: (0, 0)),
                   pl.BlockSpec((_CAST_ROWS, _HD), lambda i: (i, 0)),
                   pl.BlockSpec((_CAST_ROWS, _HD), lambda i: (i, 0))],
        out_shape=[jax.ShapeDtypeStruct((_N, _F), jnp.float32),
                   jax.ShapeDtypeStruct((1, _F), jnp.float32),
                   jax.ShapeDtypeStruct((1, _F), jnp.float32),
                   jax.ShapeDtypeStruct((_NE, _HD), jnp.int32),
                   jax.ShapeDtypeStruct((_NE, _HD), jnp.int32)],
    )(xf, W_q, down_embed, up_embed)

    keys_r = jnp.transpose(keys, (2, 0, 1, 3)).reshape(2 * _H, _NK, _KD)
    g2 = bn_gamma.reshape(1, _F)
    b2 = bn_beta.reshape(1, _F)
    idxs, ws = pl.pallas_call(
        _route_body,
        grid=(_NT,),
        in_specs=[pl.BlockSpec((_TM, _F), lambda i: (i, 0)),
                  pl.BlockSpec((1, _F), lambda i: (0, 0)),
                  pl.BlockSpec((1, _F), lambda i: (0, 0)),
                  pl.BlockSpec((1, _F), lambda i: (0, 0)),
                  pl.BlockSpec((1, _F), lambda i: (0, 0)),
                  pl.BlockSpec((2 * _H, _NK, _KD), lambda i: (0, 0, 0))],
        out_specs=[pl.BlockSpec((_TM, _PK), lambda i: (i, 0)),
                   pl.BlockSpec((_TM, _PK), lambda i: (i, 0))],
        out_shape=[jax.ShapeDtypeStruct((_N, _PK), jnp.int32),
                   jax.ShapeDtypeStruct((_N, _PK), jnp.float32)],
    )(q, s1, s2, g2, b2, keys_r)

    out = _make_sc_combine()(xf, idxs.reshape(-1), ws.reshape(-1),
                             down_pk, up_pk)
    return out.reshape(_B, _S, _D)
